# baseline (speedup 1.0000x reference)
import dataclasses
import functools

import jax
import jax.numpy as jnp
from jax import lax
from jax.experimental import pallas as pl
from jax.experimental.pallas import tpu as pltpu
from jax.experimental.pallas import tpu_sc as plsc

N = 10000
D = 128
E = 320000

NC = 2
NS = 16
NW = NC * NS
B = 128
CT0 = 113
CT1 = 45
E_PAD = NS * (CT0 + CT1) * B
NP = 10112
RPT = NP // NS


def _sc_segsum(x, src, dst, zeros_acc, with_deg):
    mesh = plsc.VectorSubcoreMesh(core_axis_name="c", subcore_axis_name="s")
    cp = pltpu.CompilerParams()
    if "needs_layout_passes" in pltpu.CompilerParams.__dataclass_fields__:
        cp = dataclasses.replace(cp, needs_layout_passes=False)

    out_type = [jax.ShapeDtypeStruct((NC * NP, D), jnp.float32)]
    scratch = [
        pltpu.VMEM((B,), jnp.int32),
        pltpu.VMEM((B,), jnp.int32),
        pltpu.VMEM((B, D), jnp.float32),
        pltpu.VMEM_SHARED((NP, D), jnp.float32),
        pltpu.SemaphoreType.DMA,
    ]
    if with_deg:
        out_type.append(jax.ShapeDtypeStruct((NW * NP,), jnp.float32))
        scratch.append(pltpu.VMEM((NP,), jnp.float32))

    @functools.partial(
        pl.kernel, mesh=mesh, out_type=out_type, scratch_types=scratch,
        compiler_params=cp)
    def run(*refs):
        if with_deg:
            (x_hbm, src_hbm, dst_hbm, zacc_hbm, out_hbm, deg_hbm,
             src_v, dst_v, rows_v, acc_sh, sem, cnt_v) = refs
        else:
            (x_hbm, src_hbm, dst_hbm, zacc_hbm,
             out_hbm, src_v, dst_v, rows_v, acc_sh, sem) = refs

        cid = lax.axis_index("c")
        sid = lax.axis_index("s")
        wid = sid * NC + cid
        r0 = sid * RPT
        base = jnp.where(cid == 0, sid * CT0 * B,
                         (NS * CT0 + sid * CT1) * B)

        if with_deg:
            z = jnp.zeros((16,), jnp.float32)

            @pl.loop(0, NP, step=16)
            def _(j):
                cnt_v[pl.ds(j, 16)] = z

        pltpu.sync_copy(zacc_hbm.at[pl.ds(r0, RPT)], acc_sh.at[pl.ds(r0, RPT)])
        plsc.subcore_barrier()

        def chunk_loop(n_chunks):
            @pl.loop(0, n_chunks)
            def _(c):
                off = base + c * B
                pltpu.sync_copy(src_hbm.at[pl.ds(off, B)], src_v)
                pltpu.sync_copy(dst_hbm.at[pl.ds(off, B)], dst_v)
                pltpu.async_copy(x_hbm.at[src_v], rows_v, sem).wait()
                pltpu.sync_copy(rows_v, acc_sh.at[dst_v], add=True)
                if with_deg:
                    one = jnp.ones((16,), jnp.float32)

                    @pl.loop(0, B, step=16)
                    def _(j):
                        idx = dst_v[pl.ds(j, 16)]
                        plsc.addupdate_scatter(cnt_v, [idx], one)

        @pl.when(cid == 0)
        def _():
            chunk_loop(CT0)

        @pl.when(cid == 1)
        def _():
            chunk_loop(CT1)

        plsc.subcore_barrier()

        pltpu.sync_copy(acc_sh.at[pl.ds(r0, RPT)],
                        out_hbm.at[pl.ds(cid * NP + r0, RPT)])
        if with_deg:
            pltpu.sync_copy(cnt_v, deg_hbm.at[pl.ds(wid * NP, NP)])

    if with_deg:
        return tuple(run(x, src, dst, zeros_acc))
    (res,) = run(x, src, dst, zeros_acc)
    return res


_dotp = functools.partial(jnp.dot, preferred_element_type=jnp.float32,
                          precision=lax.Precision.HIGHEST)
_R = 2000


def _root_mm(xin, wr_t, bias):
    def body(x_ref, wr_ref, b_ref, o_ref):
        o_ref[...] = _dotp(x_ref[...], wr_ref[...]) + b_ref[...]

    return pl.pallas_call(
        body,
        grid=(N // _R,),
        in_specs=[
            pl.BlockSpec((_R, D), lambda i: (i, 0)),
            pl.BlockSpec((D, D), lambda i: (0, 0)),
            pl.BlockSpec((1, D), lambda i: (0, 0)),
        ],
        out_specs=pl.BlockSpec((_R, D), lambda i: (i, 0)),
        out_shape=jax.ShapeDtypeStruct((N, D), jnp.float32),
    )(xin, wr_t, bias)


def _combine(sums, degp, xr, wl_t, relu):
    def body(s_ref, d_ref, xr_ref, wl_ref, o_ref):
        s = s_ref[0] + s_ref[1]
        cnt = jnp.sum(d_ref[...], axis=1)[:, None]
        mean = s / jnp.maximum(cnt, 1.0)
        acc = _dotp(mean, wl_ref[...]) + xr_ref[...]
        if relu:
            acc = jnp.maximum(acc, 0.0)
        o_ref[...] = acc

    return pl.pallas_call(
        body,
        grid=(N // _R,),
        in_specs=[
            pl.BlockSpec((2, _R, D), lambda i: (0, i, 0)),
            pl.BlockSpec((_R, NW), lambda i: (i, 0)),
            pl.BlockSpec((_R, D), lambda i: (i, 0)),
            pl.BlockSpec((D, D), lambda i: (0, 0)),
        ],
        out_specs=pl.BlockSpec((_R, D), lambda i: (i, 0)),
        out_shape=jax.ShapeDtypeStruct((N, D), jnp.float32),
    )(sums, degp, xr, wl_t)


def kernel(x, adj_t, W1l, W1r, b1, W2l, W2r, b2):
    src = adj_t[0].astype(jnp.int32)
    dst = adj_t[1].astype(jnp.int32)
    pad = E_PAD - E
    src_p = jnp.concatenate([src, jnp.zeros((pad,), jnp.int32)])
    dst_p = jnp.concatenate([dst, jnp.full((pad,), N, jnp.int32)])

    zeros_acc = jnp.zeros((NP, D), jnp.float32)

    xr1 = _root_mm(x, W1r.T, b1.reshape(1, D))
    sum1, deg = _sc_segsum(x, src_p, dst_p, zeros_acc, True)
    sum1 = sum1.reshape(NC, NP, D)
    degp = deg.reshape(NW, NP).T
    h = _combine(sum1, degp, xr1, W1l.T, relu=True)

    xr2 = _root_mm(h, W2r.T, b2.reshape(1, D))
    sum2 = _sc_segsum(h, src_p, dst_p, zeros_acc, False)
    sum2 = sum2.reshape(NC, NP, D)
    out = _combine(sum2, degp, xr2, W2l.T, relu=False)
    return out


# speedup vs baseline: 1.0326x; 1.0326x over previous
import dataclasses
import functools

import jax
import jax.numpy as jnp
from jax import lax
from jax.experimental import pallas as pl
from jax.experimental.pallas import tpu as pltpu
from jax.experimental.pallas import tpu_sc as plsc

N = 10000
D = 128
E = 320000

NC = 2
NS = 16
NW = NC * NS
B = 128
CT0 = 109
CT1 = 49
E_PAD = NS * (CT0 + CT1) * B
NP = 10112
RPT = NP // NS


def _sc_segsum(x, src, dst, zeros_acc, with_deg):
    mesh = plsc.VectorSubcoreMesh(core_axis_name="c", subcore_axis_name="s")
    cp = pltpu.CompilerParams()
    if "needs_layout_passes" in pltpu.CompilerParams.__dataclass_fields__:
        cp = dataclasses.replace(cp, needs_layout_passes=False)

    out_type = [jax.ShapeDtypeStruct((NC * NP, D), jnp.float32)]
    scratch = [
        pltpu.VMEM((B,), jnp.int32),
        pltpu.VMEM((B,), jnp.int32),
        pltpu.VMEM((B, D), jnp.float32),
        pltpu.VMEM_SHARED((NP, D), jnp.float32),
        pltpu.SemaphoreType.DMA,
    ]
    if with_deg:
        out_type.append(jax.ShapeDtypeStruct((NW * NP,), jnp.float32))
        scratch.append(pltpu.VMEM((NP,), jnp.float32))

    @functools.partial(
        pl.kernel, mesh=mesh, out_type=out_type, scratch_types=scratch,
        compiler_params=cp)
    def run(*refs):
        if with_deg:
            (x_hbm, src_hbm, dst_hbm, zacc_hbm, out_hbm, deg_hbm,
             src_v, dst_v, rows_v, acc_sh, sem, cnt_v) = refs
        else:
            (x_hbm, src_hbm, dst_hbm, zacc_hbm,
             out_hbm, src_v, dst_v, rows_v, acc_sh, sem) = refs

        cid = lax.axis_index("c")
        sid = lax.axis_index("s")
        wid = sid * NC + cid
        r0 = sid * RPT
        base = jnp.where(cid == 0, sid * CT0 * B,
                         (NS * CT0 + sid * CT1) * B)

        if with_deg:
            z = jnp.zeros((16,), jnp.float32)

            @pl.loop(0, NP, step=16)
            def _(j):
                cnt_v[pl.ds(j, 16)] = z

        pltpu.sync_copy(zacc_hbm.at[pl.ds(r0, RPT)], acc_sh.at[pl.ds(r0, RPT)])
        plsc.subcore_barrier()

        def chunk_loop(n_chunks):
            @pl.loop(0, n_chunks)
            def _(c):
                off = base + c * B
                pltpu.sync_copy(src_hbm.at[pl.ds(off, B)], src_v)
                pltpu.sync_copy(dst_hbm.at[pl.ds(off, B)], dst_v)
                pltpu.async_copy(x_hbm.at[src_v], rows_v, sem).wait()
                pltpu.sync_copy(rows_v, acc_sh.at[dst_v], add=True)
                if with_deg:
                    one = jnp.ones((16,), jnp.float32)

                    @pl.loop(0, B, step=16)
                    def _(j):
                        idx = dst_v[pl.ds(j, 16)]
                        plsc.addupdate_scatter(cnt_v, [idx], one)

        @pl.when(cid == 0)
        def _():
            chunk_loop(CT0)

        @pl.when(cid == 1)
        def _():
            chunk_loop(CT1)

        plsc.subcore_barrier()

        pltpu.sync_copy(acc_sh.at[pl.ds(r0, RPT)],
                        out_hbm.at[pl.ds(cid * NP + r0, RPT)])
        if with_deg:
            pltpu.sync_copy(cnt_v, deg_hbm.at[pl.ds(wid * NP, NP)])

    if with_deg:
        return tuple(run(x, src, dst, zeros_acc))
    (res,) = run(x, src, dst, zeros_acc)
    return res


_dotp = functools.partial(jnp.dot, preferred_element_type=jnp.float32,
                          precision=lax.Precision.HIGHEST)
_R = 2000


def _root_mm(xin, wr_t, bias):
    def body(x_ref, wr_ref, b_ref, o_ref):
        o_ref[...] = _dotp(x_ref[...], wr_ref[...]) + b_ref[...]

    return pl.pallas_call(
        body,
        grid=(N // _R,),
        in_specs=[
            pl.BlockSpec((_R, D), lambda i: (i, 0)),
            pl.BlockSpec((D, D), lambda i: (0, 0)),
            pl.BlockSpec((1, D), lambda i: (0, 0)),
        ],
        out_specs=pl.BlockSpec((_R, D), lambda i: (i, 0)),
        out_shape=jax.ShapeDtypeStruct((N, D), jnp.float32),
    )(xin, wr_t, bias)


def _combine(sums, degp, xr, wl_t, relu):
    def body(s_ref, d_ref, xr_ref, wl_ref, o_ref):
        s = s_ref[0] + s_ref[1]
        cnt = jnp.sum(d_ref[...], axis=1)[:, None]
        mean = s / jnp.maximum(cnt, 1.0)
        acc = _dotp(mean, wl_ref[...]) + xr_ref[...]
        if relu:
            acc = jnp.maximum(acc, 0.0)
        o_ref[...] = acc

    return pl.pallas_call(
        body,
        grid=(N // _R,),
        in_specs=[
            pl.BlockSpec((2, _R, D), lambda i: (0, i, 0)),
            pl.BlockSpec((_R, NW), lambda i: (i, 0)),
            pl.BlockSpec((_R, D), lambda i: (i, 0)),
            pl.BlockSpec((D, D), lambda i: (0, 0)),
        ],
        out_specs=pl.BlockSpec((_R, D), lambda i: (i, 0)),
        out_shape=jax.ShapeDtypeStruct((N, D), jnp.float32),
    )(sums, degp, xr, wl_t)


def kernel(x, adj_t, W1l, W1r, b1, W2l, W2r, b2):
    src = adj_t[0].astype(jnp.int32)
    dst = adj_t[1].astype(jnp.int32)
    pad = E_PAD - E
    src_p = jnp.concatenate([src, jnp.zeros((pad,), jnp.int32)])
    dst_p = jnp.concatenate([dst, jnp.full((pad,), N, jnp.int32)])

    zeros_acc = jnp.zeros((NP, D), jnp.float32)

    xr1 = _root_mm(x, W1r.T, b1.reshape(1, D))
    sum1, deg = _sc_segsum(x, src_p, dst_p, zeros_acc, True)
    sum1 = sum1.reshape(NC, NP, D)
    degp = deg.reshape(NW, NP).T
    h = _combine(sum1, degp, xr1, W1l.T, relu=True)

    xr2 = _root_mm(h, W2r.T, b2.reshape(1, D))
    sum2 = _sc_segsum(h, src_p, dst_p, zeros_acc, False)
    sum2 = sum2.reshape(NC, NP, D)
    out = _combine(sum2, degp, xr2, W2l.T, relu=False)
    return out


# speedup vs baseline: 1.0494x; 1.0163x over previous
import dataclasses
import functools

import jax
import jax.numpy as jnp
from jax import lax
from jax.experimental import pallas as pl
from jax.experimental.pallas import tpu as pltpu
from jax.experimental.pallas import tpu_sc as plsc

N = 10000
D = 128
E = 320000

NC = 2
NS = 16
NW = NC * NS
B = 128
CT0 = 105
CT1 = 53
E_PAD = NS * (CT0 + CT1) * B
NP = 10112
RPT = NP // NS


def _sc_segsum(x, src, dst, zeros_acc, with_deg):
    mesh = plsc.VectorSubcoreMesh(core_axis_name="c", subcore_axis_name="s")
    cp = pltpu.CompilerParams()
    if "needs_layout_passes" in pltpu.CompilerParams.__dataclass_fields__:
        cp = dataclasses.replace(cp, needs_layout_passes=False)

    out_type = [jax.ShapeDtypeStruct((NC * NP, D), jnp.float32)]
    scratch = [
        pltpu.VMEM((B,), jnp.int32),
        pltpu.VMEM((B,), jnp.int32),
        pltpu.VMEM((B, D), jnp.float32),
        pltpu.VMEM_SHARED((NP, D), jnp.float32),
        pltpu.SemaphoreType.DMA,
    ]
    if with_deg:
        out_type.append(jax.ShapeDtypeStruct((NW * NP,), jnp.float32))
        scratch.append(pltpu.VMEM((NP,), jnp.float32))

    @functools.partial(
        pl.kernel, mesh=mesh, out_type=out_type, scratch_types=scratch,
        compiler_params=cp)
    def run(*refs):
        if with_deg:
            (x_hbm, src_hbm, dst_hbm, zacc_hbm, out_hbm, deg_hbm,
             src_v, dst_v, rows_v, acc_sh, sem, cnt_v) = refs
        else:
            (x_hbm, src_hbm, dst_hbm, zacc_hbm,
             out_hbm, src_v, dst_v, rows_v, acc_sh, sem) = refs

        cid = lax.axis_index("c")
        sid = lax.axis_index("s")
        wid = sid * NC + cid
        r0 = sid * RPT
        base = jnp.where(cid == 0, sid * CT0 * B,
                         (NS * CT0 + sid * CT1) * B)

        if with_deg:
            z = jnp.zeros((16,), jnp.float32)

            @pl.loop(0, NP, step=16)
            def _(j):
                cnt_v[pl.ds(j, 16)] = z

        pltpu.sync_copy(zacc_hbm.at[pl.ds(r0, RPT)], acc_sh.at[pl.ds(r0, RPT)])
        plsc.subcore_barrier()

        def chunk_loop(n_chunks):
            @pl.loop(0, n_chunks)
            def _(c):
                off = base + c * B
                pltpu.sync_copy(src_hbm.at[pl.ds(off, B)], src_v)
                pltpu.sync_copy(dst_hbm.at[pl.ds(off, B)], dst_v)
                pltpu.async_copy(x_hbm.at[src_v], rows_v, sem).wait()
                pltpu.sync_copy(rows_v, acc_sh.at[dst_v], add=True)
                if with_deg:
                    one = jnp.ones((16,), jnp.float32)

                    @pl.loop(0, B, step=16)
                    def _(j):
                        idx = dst_v[pl.ds(j, 16)]
                        plsc.addupdate_scatter(cnt_v, [idx], one)

        @pl.when(cid == 0)
        def _():
            chunk_loop(CT0)

        @pl.when(cid == 1)
        def _():
            chunk_loop(CT1)

        plsc.subcore_barrier()

        pltpu.sync_copy(acc_sh.at[pl.ds(r0, RPT)],
                        out_hbm.at[pl.ds(cid * NP + r0, RPT)])
        if with_deg:
            pltpu.sync_copy(cnt_v, deg_hbm.at[pl.ds(wid * NP, NP)])

    if with_deg:
        return tuple(run(x, src, dst, zeros_acc))
    (res,) = run(x, src, dst, zeros_acc)
    return res


_dotp = functools.partial(jnp.dot, preferred_element_type=jnp.float32,
                          precision=lax.Precision.HIGHEST)
_R = 2000


def _root_mm(xin, wr_t, bias):
    def body(x_ref, wr_ref, b_ref, o_ref):
        o_ref[...] = _dotp(x_ref[...], wr_ref[...]) + b_ref[...]

    return pl.pallas_call(
        body,
        grid=(N // _R,),
        in_specs=[
            pl.BlockSpec((_R, D), lambda i: (i, 0)),
            pl.BlockSpec((D, D), lambda i: (0, 0)),
            pl.BlockSpec((1, D), lambda i: (0, 0)),
        ],
        out_specs=pl.BlockSpec((_R, D), lambda i: (i, 0)),
        out_shape=jax.ShapeDtypeStruct((N, D), jnp.float32),
    )(xin, wr_t, bias)


def _combine(sums, degp, xr, wl_t, relu):
    def body(s_ref, d_ref, xr_ref, wl_ref, o_ref):
        s = s_ref[0] + s_ref[1]
        cnt = jnp.sum(d_ref[...], axis=1)[:, None]
        mean = s / jnp.maximum(cnt, 1.0)
        acc = _dotp(mean, wl_ref[...]) + xr_ref[...]
        if relu:
            acc = jnp.maximum(acc, 0.0)
        o_ref[...] = acc

    return pl.pallas_call(
        body,
        grid=(N // _R,),
        in_specs=[
            pl.BlockSpec((2, _R, D), lambda i: (0, i, 0)),
            pl.BlockSpec((_R, NW), lambda i: (i, 0)),
            pl.BlockSpec((_R, D), lambda i: (i, 0)),
            pl.BlockSpec((D, D), lambda i: (0, 0)),
        ],
        out_specs=pl.BlockSpec((_R, D), lambda i: (i, 0)),
        out_shape=jax.ShapeDtypeStruct((N, D), jnp.float32),
    )(sums, degp, xr, wl_t)


def kernel(x, adj_t, W1l, W1r, b1, W2l, W2r, b2):
    src = adj_t[0].astype(jnp.int32)
    dst = adj_t[1].astype(jnp.int32)
    pad = E_PAD - E
    src_p = jnp.concatenate([src, jnp.zeros((pad,), jnp.int32)])
    dst_p = jnp.concatenate([dst, jnp.full((pad,), N, jnp.int32)])

    zeros_acc = jnp.zeros((NP, D), jnp.float32)

    xr1 = _root_mm(x, W1r.T, b1.reshape(1, D))
    sum1, deg = _sc_segsum(x, src_p, dst_p, zeros_acc, True)
    sum1 = sum1.reshape(NC, NP, D)
    degp = deg.reshape(NW, NP).T
    h = _combine(sum1, degp, xr1, W1l.T, relu=True)

    xr2 = _root_mm(h, W2r.T, b2.reshape(1, D))
    sum2 = _sc_segsum(h, src_p, dst_p, zeros_acc, False)
    sum2 = sum2.reshape(NC, NP, D)
    out = _combine(sum2, degp, xr2, W2l.T, relu=False)
    return out
